# baseline (speedup 1.0000x reference)
import functools

import jax
import jax.numpy as jnp
from jax import lax
from jax.experimental import pallas as pl
from jax.experimental.pallas import tpu as pltpu
from jax.experimental.pallas import tpu_sc as plsc

B, NB, PL, H, F = 16, 64, 24, 1024, 4096
M = B * NB
NC, NS, L = 2, 16, 16
NW = NC * NS
ROWS = M // NW
CH = H // L

BMV = 256
BML = 256


def _sc_pool_body(lang_hbm, out_hbm, buf, obuf, sem0, sem1):
    wid = lax.axis_index("s") * NC + lax.axis_index("c")
    base = wid * ROWS
    sems = (sem0, sem1)

    pltpu.async_copy(lang_hbm.at[base + 0], buf.at[0], sem0)
    pltpu.async_copy(lang_hbm.at[base + 1], buf.at[1], sem1)

    def row_pair(i, carry):
        r0 = 2 * i
        for b in range(2):
            r = r0 + b
            pltpu.make_async_copy(lang_hbm.at[base + r], buf.at[b], sems[b]).wait()

            def chunk(c, carry2):
                off = c * L
                acc = buf[b, 0, pl.ds(off, L)]
                for t in range(1, PL):
                    acc = acc + buf[b, t, pl.ds(off, L)]
                obuf[r, pl.ds(off, L)] = acc
                return carry2

            lax.fori_loop(0, CH, chunk, 0)

            @pl.when(r + 2 < ROWS)
            def _():
                pltpu.async_copy(lang_hbm.at[base + r + 2], buf.at[b], sems[b])

        return carry

    lax.fori_loop(0, ROWS // 2, row_pair, 0)
    pltpu.sync_copy(obuf, out_hbm.at[pl.ds(base, ROWS)])


_sc_pool = functools.partial(
    pl.kernel,
    out_type=jax.ShapeDtypeStruct((M, H), jnp.float32),
    mesh=plsc.VectorSubcoreMesh(core_axis_name="c", subcore_axis_name="s",
                                num_cores=NC, num_subcores=NS),
    scratch_types=[
        pltpu.VMEM((2, PL, H), jnp.float32),
        pltpu.VMEM((ROWS, H), jnp.float32),
        pltpu.SemaphoreType.DMA,
        pltpu.SemaphoreType.DMA,
    ],
)(_sc_pool_body)


def _vis_body(vis_ref, wv_ref, bv_ref, out_ref):
    out_ref[...] = (
        lax.dot_general(vis_ref[...], wv_ref[...], (((1,), (1,)), ((), ())),
                        preferred_element_type=jnp.float32)
        + bv_ref[...]
    )


def _lang_body(pooled_ref, invlen_ref, wl_ref, bl_ref, out_ref):
    scaled = pooled_ref[...] * invlen_ref[...]
    out_ref[...] = (
        lax.dot_general(scaled, wl_ref[...], (((1,), (1,)), ((), ())),
                        preferred_element_type=jnp.float32)
        + bl_ref[...]
    )


@functools.partial(jax.jit, static_argnames=())
def kernel(vision, language, phrase_lengths, Wv, bv, Wl, bl):
    vis = vision.reshape(M, F)
    lang = language.reshape(M, PL, H)
    inv_len = (1.0 / phrase_lengths.astype(jnp.float32)).reshape(M, 1)

    pooled = _sc_pool(lang)

    vmap = pl.pallas_call(
        _vis_body,
        grid=(M // BMV,),
        in_specs=[
            pl.BlockSpec((BMV, F), lambda i: (i, 0)),
            pl.BlockSpec((H, F), lambda i: (0, 0)),
            pl.BlockSpec((1, H), lambda i: (0, 0)),
        ],
        out_specs=pl.BlockSpec((BMV, H), lambda i: (i, 0)),
        out_shape=jax.ShapeDtypeStruct((M, H), jnp.float32),
    )(vis, Wv, bv.reshape(1, H))

    lmap = pl.pallas_call(
        _lang_body,
        grid=(M // BML,),
        in_specs=[
            pl.BlockSpec((BML, H), lambda i: (i, 0)),
            pl.BlockSpec((BML, 1), lambda i: (i, 0)),
            pl.BlockSpec((H, H), lambda i: (0, 0)),
            pl.BlockSpec((1, H), lambda i: (0, 0)),
        ],
        out_specs=pl.BlockSpec((BML, H), lambda i: (i, 0)),
        out_shape=jax.ShapeDtypeStruct((M, H), jnp.float32),
    )(pooled, inv_len, Wl, bl.reshape(1, H))

    return (lmap.reshape(B, NB, H), vmap.reshape(B, NB, H))
